# Initial kernel scaffold; baseline (speedup 1.0000x reference)
import jax
import jax.numpy as jnp
from jax.experimental import pallas as pl


def kernel(router_input, hidden_states, Wr, w13, w2):
    raise NotImplementedError("write your pallas kernel here")


# speedup vs baseline: 1.3601x; 1.3601x over previous
import functools

import jax
import jax.numpy as jnp
from jax import lax
from jax.experimental import pallas as pl
from jax.experimental.pallas import tpu as pltpu
from jax.experimental.pallas import tpu_sc as plsc

T = 2048
D = 1024
F = 1024
E = 8
K = 2

TM = 256
R_CAP = T * K + E * TM
NT = R_CAP // TM

NC, NS = 2, 16
NW = NC * NS
TPW = T // NW
NG = TPW // 16

_MESH = plsc.VectorSubcoreMesh(core_axis_name="c", subcore_axis_name="s")
_SC_PARAMS = pltpu.CompilerParams(needs_layout_passes=False)


def _lane_iota():
    return lax.iota(jnp.int32, 16)


def _router_body(wr_ref, x_ref, out_ref):
    out_ref[...] = lax.dot_general(
        wr_ref[...], x_ref[...], (((1,), (1,)), ((), ())),
        preferred_element_type=jnp.float32)


def _router(wr, x):
    return pl.pallas_call(
        _router_body,
        out_shape=jax.ShapeDtypeStruct((E, T), jnp.float32),
    )(wr, x)


def _route_body(logits_hbm, eids_hbm, wts_hbm, hist_hbm, lbuf, ebuf0, ebuf1,
                wbuf0, wbuf1, hbuf):
    wid = lax.axis_index("s") * NC + lax.axis_index("c")
    base = wid * TPW
    for e in range(E):
        pltpu.sync_copy(logits_hbm.at[e, pl.ds(base, TPW)], lbuf.at[e])
    lane = _lane_iota()
    hacc = jnp.zeros((16,), jnp.int32)
    for g in range(NG):
        v = [lbuf[e, pl.ds(g * 16, 16)] for e in range(E)]
        m1 = v[0]
        i1 = jnp.zeros((16,), jnp.int32)
        for e in range(1, E):
            upd = v[e] > m1
            m1 = jnp.where(upd, v[e], m1)
            i1 = jnp.where(upd, jnp.int32(e), i1)
        m2 = jnp.full((16,), -jnp.inf, jnp.float32)
        i2 = jnp.zeros((16,), jnp.int32)
        for e in range(E):
            upd = (i1 != e) & (v[e] > m2)
            m2 = jnp.where(upd, v[e], m2)
            i2 = jnp.where(upd, jnp.int32(e), i2)
        dd = jnp.exp(m2 - m1)
        denom = 1.0 + dd
        ebuf0[pl.ds(g * 16, 16)] = i1
        ebuf1[pl.ds(g * 16, 16)] = i2
        wbuf0[pl.ds(g * 16, 16)] = 1.0 / denom
        wbuf1[pl.ds(g * 16, 16)] = dd / denom
        for e in range(E):
            c1 = plsc.all_reduce_population_count(i1 == e)
            c2 = plsc.all_reduce_population_count(i2 == e)
            hacc = hacc + jnp.where(lane == e, c1 + c2, 0)
    hbuf[...] = hacc
    pltpu.sync_copy(ebuf0, eids_hbm.at[pl.ds(base, TPW)])
    pltpu.sync_copy(ebuf1, eids_hbm.at[pl.ds(T + base, TPW)])
    pltpu.sync_copy(wbuf0, wts_hbm.at[pl.ds(base, TPW)])
    pltpu.sync_copy(wbuf1, wts_hbm.at[pl.ds(T + base, TPW)])
    pltpu.sync_copy(hbuf, hist_hbm.at[wid])


_route = functools.partial(
    pl.kernel, _route_body, mesh=_MESH, compiler_params=_SC_PARAMS,
    out_type=[
        jax.ShapeDtypeStruct((K * T,), jnp.int32),
        jax.ShapeDtypeStruct((K * T,), jnp.float32),
        jax.ShapeDtypeStruct((NW, 16), jnp.int32),
    ],
    scratch_types=[
        pltpu.VMEM((E, TPW), jnp.float32),
        pltpu.VMEM((TPW,), jnp.int32),
        pltpu.VMEM((TPW,), jnp.int32),
        pltpu.VMEM((TPW,), jnp.float32),
        pltpu.VMEM((TPW,), jnp.float32),
        pltpu.VMEM((16,), jnp.int32),
    ],
)()


def _dispatch_body(eids_hbm, wts_hbm, hist_hbm, hidden_hbm,
                   xs_hbm, ws_hbm, pos_hbm, te_hbm, tv_hbm,
                   histv, ebuf0, ebuf1, wbuf0, wbuf1, pbuf0, pbuf1,
                   rows_v, tebuf, tvbuf, sem):
    wid = lax.axis_index("s") * NC + lax.axis_index("c")
    base = wid * TPW
    lane = _lane_iota()

    pltpu.sync_copy(hist_hbm, histv)
    tot = jnp.zeros((16,), jnp.int32)
    woff = jnp.zeros((16,), jnp.int32)
    for w in range(NW):
        row = histv[w, :]
        woff = jnp.where(w < wid, woff + row, woff)
        tot = tot + row
    pad_tot = ((tot + (TM - 1)) >> 8) << 8
    gbase = plsc.cumsum(pad_tot) - pad_tot
    bw = gbase + woff

    bw_s = [jnp.sum(jnp.where(lane == e, bw, 0)) for e in range(E)]
    base_s = [jnp.sum(jnp.where(lane == e, gbase, 0)) for e in range(E)]
    tot_s = [jnp.sum(jnp.where(lane == e, tot, 0)) for e in range(E)]

    pltpu.sync_copy(eids_hbm.at[pl.ds(base, TPW)], ebuf0)
    pltpu.sync_copy(eids_hbm.at[pl.ds(T + base, TPW)], ebuf1)
    pltpu.sync_copy(wts_hbm.at[pl.ds(base, TPW)], wbuf0)
    pltpu.sync_copy(wts_hbm.at[pl.ds(T + base, TPW)], wbuf1)

    carry = [jnp.zeros((16,), jnp.int32) for _ in range(E)]
    for ebuf, pbuf in ((ebuf0, pbuf0), (ebuf1, pbuf1)):
        for g in range(NG):
            evec = ebuf[pl.ds(g * 16, 16)]
            dst = jnp.zeros((16,), jnp.int32)
            for e in range(E):
                mask = evec == e
                rank = plsc.cumsum(mask.astype(jnp.int32)) - 1 + carry[e]
                dst = jnp.where(mask, bw_s[e] + rank, dst)
                carry[e] = carry[e] + plsc.all_reduce_population_count(mask)
            pbuf[pl.ds(g * 16, 16)] = dst

    pltpu.sync_copy(pbuf0, pos_hbm.at[pl.ds(base, TPW)])
    pltpu.sync_copy(pbuf1, pos_hbm.at[pl.ds(T + base, TPW)])

    pltpu.sync_copy(hidden_hbm.at[pl.ds(base, TPW)], rows_v)
    pltpu.async_copy(rows_v, xs_hbm.at[pbuf0], sem).wait()
    pltpu.async_copy(rows_v, xs_hbm.at[pbuf1], sem).wait()
    pltpu.async_copy(wbuf0, ws_hbm.at[pbuf0], sem).wait()
    pltpu.async_copy(wbuf1, ws_hbm.at[pbuf1], sem).wait()

    @pl.when(wid == 0)
    def _():
        total_rows = jnp.sum(jnp.where(lane < E, pad_tot, 0))
        row_last = total_rows - 1
        c_last = jnp.zeros((), jnp.int32)
        for e in range(E):
            c_last = c_last + jnp.where(base_s[e] <= row_last, 1, 0)
        e_last = c_last - 1
        for grp in range(NT // 16 + (1 if NT % 16 else 0)):
            jv = lane + grp * 16
            rowv = jv * TM
            cj = jnp.zeros((16,), jnp.int32)
            for e in range(E):
                cj = cj + jnp.where(base_s[e] <= rowv, 1, 0)
            ej = cj - 1
            endv = jnp.zeros((16,), jnp.int32)
            for e in range(E):
                endv = endv + jnp.where(ej == e, base_s[e] + tot_s[e], 0)
            validv = rowv < endv
            tev = jnp.where(validv, ej, e_last)
            tebuf[pl.ds(grp * 16, 16)] = tev
            tvbuf[pl.ds(grp * 16, 16)] = validv.astype(jnp.int32)
        pltpu.sync_copy(tebuf, te_hbm)
        pltpu.sync_copy(tvbuf, tv_hbm)


_dispatch = functools.partial(
    pl.kernel, _dispatch_body, mesh=_MESH, compiler_params=_SC_PARAMS,
    out_type=[
        jax.ShapeDtypeStruct((R_CAP, D), jnp.float32),
        jax.ShapeDtypeStruct((R_CAP,), jnp.float32),
        jax.ShapeDtypeStruct((K * T,), jnp.int32),
        jax.ShapeDtypeStruct((32,), jnp.int32),
        jax.ShapeDtypeStruct((32,), jnp.int32),
    ],
    scratch_types=[
        pltpu.VMEM((NW, 16), jnp.int32),
        pltpu.VMEM((TPW,), jnp.int32),
        pltpu.VMEM((TPW,), jnp.int32),
        pltpu.VMEM((TPW,), jnp.float32),
        pltpu.VMEM((TPW,), jnp.float32),
        pltpu.VMEM((TPW,), jnp.int32),
        pltpu.VMEM((TPW,), jnp.int32),
        pltpu.VMEM((TPW, D), jnp.float32),
        pltpu.VMEM((32,), jnp.int32),
        pltpu.VMEM((32,), jnp.int32),
        pltpu.SemaphoreType.DMA,
    ],
)()


def _ffn_body(te_ref, tv_ref, x_ref, w13_ref, w2_ref, ws_ref, y_ref):
    i = pl.program_id(0)

    @pl.when(tv_ref[i] == 1)
    def _():
        xb = x_ref[...]
        g = lax.dot_general(xb, w13_ref[0, :F, :], (((1,), (1,)), ((), ())),
                            preferred_element_type=jnp.float32)
        u = lax.dot_general(xb, w13_ref[0, F:, :], (((1,), (1,)), ((), ())),
                            preferred_element_type=jnp.float32)
        act = jnp.maximum(g, 0.0) * u
        y = lax.dot_general(act, w2_ref[0], (((1,), (1,)), ((), ())),
                            preferred_element_type=jnp.float32)
        y_ref[...] = y * ws_ref[0, 0, :][:, None]


def _ffn(tile_e, tile_v, x_sorted, w13, w2, w_sorted):
    return pl.pallas_call(
        _ffn_body,
        grid_spec=pltpu.PrefetchScalarGridSpec(
            num_scalar_prefetch=2,
            grid=(NT,),
            in_specs=[
                pl.BlockSpec((TM, D), lambda i, te, tv: (i, 0)),
                pl.BlockSpec((1, 2 * F, D), lambda i, te, tv: (te[i], 0, 0)),
                pl.BlockSpec((1, D, F), lambda i, te, tv: (te[i], 0, 0)),
                pl.BlockSpec((1, 1, TM), lambda i, te, tv: (i, 0, 0)),
            ],
            out_specs=pl.BlockSpec((TM, D), lambda i, te, tv: (i, 0)),
        ),
        out_shape=jax.ShapeDtypeStruct((R_CAP, D), jnp.float32),
        compiler_params=pltpu.CompilerParams(
            dimension_semantics=("arbitrary",)),
    )(tile_e, tile_v, x_sorted, w13, w2, w_sorted)


_DCHUNK = 16


def _combine_body(y_hbm, pos_hbm, out_hbm, idx0, idx1, r0, r1, sem):
    wid = lax.axis_index("s") * NC + lax.axis_index("c")
    base = wid * TPW

    def chunk(c, _):
        off = base + c * _DCHUNK
        pltpu.sync_copy(pos_hbm.at[pl.ds(off, _DCHUNK)], idx0)
        pltpu.sync_copy(pos_hbm.at[pl.ds(T + off, _DCHUNK)], idx1)
        pltpu.async_copy(y_hbm.at[idx0], r0, sem).wait()
        pltpu.async_copy(y_hbm.at[idx1], r1, sem).wait()
        for t in range(_DCHUNK):
            for s in range(D // 16):
                sl = pl.ds(s * 16, 16)
                r0[t, sl] = r0[t, sl] + r1[t, sl]
        pltpu.sync_copy(r0, out_hbm.at[pl.ds(off, _DCHUNK)])
        return _

    lax.fori_loop(0, TPW // _DCHUNK, chunk, 0)


_combine = functools.partial(
    pl.kernel, _combine_body, mesh=_MESH, compiler_params=_SC_PARAMS,
    out_type=jax.ShapeDtypeStruct((T, D), jnp.float32),
    scratch_types=[
        pltpu.VMEM((_DCHUNK,), jnp.int32),
        pltpu.VMEM((_DCHUNK,), jnp.int32),
        pltpu.VMEM((_DCHUNK, D), jnp.float32),
        pltpu.VMEM((_DCHUNK, D), jnp.float32),
        pltpu.SemaphoreType.DMA,
    ],
)()


def kernel(router_input, hidden_states, Wr, w13, w2):
    logits = _router(Wr, router_input)
    eids, wts, hist = _route(logits)
    x_sorted, w_sorted, pos, tile_e, tile_v = _dispatch(
        eids, wts, hist, hidden_states)
    y = _ffn(tile_e, tile_v, x_sorted, w13, w2,
             w_sorted.reshape(NT, 1, TM))
    return _combine(y, pos)


# speedup vs baseline: 1.6078x; 1.1822x over previous
import functools

import jax
import jax.numpy as jnp
from jax import lax
from jax.experimental import pallas as pl
from jax.experimental.pallas import tpu as pltpu
from jax.experimental.pallas import tpu_sc as plsc

T = 2048
D = 1024
F = 1024
E = 8
K = 2

TM = 256
R_CAP = T * K + E * TM
NT = R_CAP // TM

NC, NS = 2, 16
NW = NC * NS
TPW = T // NW
NG = TPW // 16

_MESH = plsc.VectorSubcoreMesh(core_axis_name="c", subcore_axis_name="s")
_SC_PARAMS = pltpu.CompilerParams(needs_layout_passes=False)


def _lane_iota():
    return lax.iota(jnp.int32, 16)


def _router_body(wr_ref, x_ref, out_ref):
    out_ref[...] = lax.dot_general(
        wr_ref[...], x_ref[...], (((1,), (1,)), ((), ())),
        preferred_element_type=jnp.float32)


def _router(wr, x):
    return pl.pallas_call(
        _router_body,
        out_shape=jax.ShapeDtypeStruct((E, T), jnp.float32),
    )(wr, x)


def _route_body(logits_hbm, eids_hbm, wts_hbm, hist_hbm, lbuf, ebuf0, ebuf1,
                wbuf0, wbuf1, hbuf):
    wid = lax.axis_index("s") * NC + lax.axis_index("c")
    base = wid * TPW
    for e in range(E):
        pltpu.sync_copy(logits_hbm.at[e, pl.ds(base, TPW)], lbuf.at[e])
    lane = _lane_iota()
    hacc = jnp.zeros((16,), jnp.int32)
    for g in range(NG):
        v = [lbuf[e, pl.ds(g * 16, 16)] for e in range(E)]
        m1 = v[0]
        i1 = jnp.zeros((16,), jnp.int32)
        for e in range(1, E):
            upd = v[e] > m1
            m1 = jnp.where(upd, v[e], m1)
            i1 = jnp.where(upd, jnp.int32(e), i1)
        m2 = jnp.full((16,), -jnp.inf, jnp.float32)
        i2 = jnp.zeros((16,), jnp.int32)
        for e in range(E):
            upd = (i1 != e) & (v[e] > m2)
            m2 = jnp.where(upd, v[e], m2)
            i2 = jnp.where(upd, jnp.int32(e), i2)
        dd = jnp.exp(m2 - m1)
        denom = 1.0 + dd
        ebuf0[pl.ds(g * 16, 16)] = i1
        ebuf1[pl.ds(g * 16, 16)] = i2
        wbuf0[pl.ds(g * 16, 16)] = 1.0 / denom
        wbuf1[pl.ds(g * 16, 16)] = dd / denom
        for e in range(E):
            c1 = plsc.all_reduce_population_count(i1 == e)
            c2 = plsc.all_reduce_population_count(i2 == e)
            hacc = hacc + jnp.where(lane == e, c1 + c2, 0)
    hbuf[...] = hacc
    pltpu.sync_copy(ebuf0, eids_hbm.at[pl.ds(base, TPW)])
    pltpu.sync_copy(ebuf1, eids_hbm.at[pl.ds(T + base, TPW)])
    pltpu.sync_copy(wbuf0, wts_hbm.at[pl.ds(base, TPW)])
    pltpu.sync_copy(wbuf1, wts_hbm.at[pl.ds(T + base, TPW)])
    pltpu.sync_copy(hbuf, hist_hbm.at[wid])


_route = functools.partial(
    pl.kernel, _route_body, mesh=_MESH, compiler_params=_SC_PARAMS,
    out_type=[
        jax.ShapeDtypeStruct((K * T,), jnp.int32),
        jax.ShapeDtypeStruct((K * T,), jnp.float32),
        jax.ShapeDtypeStruct((NW, 16), jnp.int32),
    ],
    scratch_types=[
        pltpu.VMEM((E, TPW), jnp.float32),
        pltpu.VMEM((TPW,), jnp.int32),
        pltpu.VMEM((TPW,), jnp.int32),
        pltpu.VMEM((TPW,), jnp.float32),
        pltpu.VMEM((TPW,), jnp.float32),
        pltpu.VMEM((16,), jnp.int32),
    ],
)()


def _dispatch_body(eids_hbm, wts_hbm, hist_hbm, hidden_hbm,
                   xs_hbm, ws_hbm, pos_hbm, te_hbm, tv_hbm,
                   histv, ebuf0, ebuf1, wbuf0, wbuf1, pbuf0, pbuf1,
                   rows_v, tebuf, tvbuf, sem_h, sem_e, sem_r, sem_s, sem_p):
    wid = lax.axis_index("s") * NC + lax.axis_index("c")
    base = wid * TPW
    lane = _lane_iota()

    cp_h = pltpu.async_copy(hist_hbm, histv, sem_h)
    cp_e0 = pltpu.async_copy(eids_hbm.at[pl.ds(base, TPW)], ebuf0, sem_e)
    cp_e1 = pltpu.async_copy(eids_hbm.at[pl.ds(T + base, TPW)], ebuf1, sem_e)
    cp_w0 = pltpu.async_copy(wts_hbm.at[pl.ds(base, TPW)], wbuf0, sem_e)
    cp_w1 = pltpu.async_copy(wts_hbm.at[pl.ds(T + base, TPW)], wbuf1, sem_e)
    cp_r = pltpu.async_copy(hidden_hbm.at[pl.ds(base, TPW)], rows_v, sem_r)

    cp_h.wait()
    tot = jnp.zeros((16,), jnp.int32)
    woff = jnp.zeros((16,), jnp.int32)
    for w in range(NW):
        row = histv[w, :]
        woff = jnp.where(w < wid, woff + row, woff)
        tot = tot + row
    pad_tot = ((tot + (TM - 1)) >> 8) << 8
    gbase = plsc.cumsum(pad_tot) - pad_tot
    bw = gbase + woff

    bw_s = [jnp.sum(jnp.where(lane == e, bw, 0)) for e in range(E)]
    base_s = [jnp.sum(jnp.where(lane == e, gbase, 0)) for e in range(E)]
    tot_s = [jnp.sum(jnp.where(lane == e, tot, 0)) for e in range(E)]

    cp_e0.wait()
    cp_e1.wait()
    carry = [jnp.zeros((16,), jnp.int32) for _ in range(E)]
    for ebuf, pbuf in ((ebuf0, pbuf0), (ebuf1, pbuf1)):
        for g in range(NG):
            evec = ebuf[pl.ds(g * 16, 16)]
            dst = jnp.zeros((16,), jnp.int32)
            for e in range(E):
                mask = evec == e
                rank = plsc.cumsum(mask.astype(jnp.int32)) - 1 + carry[e]
                dst = jnp.where(mask, bw_s[e] + rank, dst)
                carry[e] = carry[e] + plsc.all_reduce_population_count(mask)
            pbuf[pl.ds(g * 16, 16)] = dst

    cp_p0 = pltpu.async_copy(pbuf0, pos_hbm.at[pl.ds(base, TPW)], sem_p)
    cp_p1 = pltpu.async_copy(pbuf1, pos_hbm.at[pl.ds(T + base, TPW)], sem_p)

    cp_r.wait()
    cp_w0.wait()
    cp_w1.wait()
    cp_s0 = pltpu.async_copy(rows_v, xs_hbm.at[pbuf0], sem_s)
    cp_s1 = pltpu.async_copy(rows_v, xs_hbm.at[pbuf1], sem_s)
    cp_s2 = pltpu.async_copy(wbuf0, ws_hbm.at[pbuf0], sem_s)
    cp_s3 = pltpu.async_copy(wbuf1, ws_hbm.at[pbuf1], sem_s)

    @pl.when(wid == 0)
    def _():
        total_rows = jnp.sum(jnp.where(lane < E, pad_tot, 0))
        row_last = total_rows - 1
        c_last = jnp.zeros((), jnp.int32)
        for e in range(E):
            c_last = c_last + jnp.where(base_s[e] <= row_last, 1, 0)
        e_last = c_last - 1
        for grp in range(NT // 16 + (1 if NT % 16 else 0)):
            jv = lane + grp * 16
            rowv = jv * TM
            cj = jnp.zeros((16,), jnp.int32)
            for e in range(E):
                cj = cj + jnp.where(base_s[e] <= rowv, 1, 0)
            ej = cj - 1
            endv = jnp.zeros((16,), jnp.int32)
            for e in range(E):
                endv = endv + jnp.where(ej == e, base_s[e] + tot_s[e], 0)
            validv = rowv < endv
            tev = jnp.where(validv, ej, e_last)
            tebuf[pl.ds(grp * 16, 16)] = tev
            tvbuf[pl.ds(grp * 16, 16)] = validv.astype(jnp.int32)
        pltpu.sync_copy(tebuf, te_hbm)
        pltpu.sync_copy(tvbuf, tv_hbm)

    cp_p0.wait()
    cp_p1.wait()
    cp_s0.wait()
    cp_s1.wait()
    cp_s2.wait()
    cp_s3.wait()


_dispatch = functools.partial(
    pl.kernel, _dispatch_body, mesh=_MESH, compiler_params=_SC_PARAMS,
    out_type=[
        jax.ShapeDtypeStruct((R_CAP, D), jnp.float32),
        jax.ShapeDtypeStruct((R_CAP,), jnp.float32),
        jax.ShapeDtypeStruct((K * T,), jnp.int32),
        jax.ShapeDtypeStruct((32,), jnp.int32),
        jax.ShapeDtypeStruct((32,), jnp.int32),
    ],
    scratch_types=[
        pltpu.VMEM((NW, 16), jnp.int32),
        pltpu.VMEM((TPW,), jnp.int32),
        pltpu.VMEM((TPW,), jnp.int32),
        pltpu.VMEM((TPW,), jnp.float32),
        pltpu.VMEM((TPW,), jnp.float32),
        pltpu.VMEM((TPW,), jnp.int32),
        pltpu.VMEM((TPW,), jnp.int32),
        pltpu.VMEM((TPW, D), jnp.float32),
        pltpu.VMEM((32,), jnp.int32),
        pltpu.VMEM((32,), jnp.int32),
        pltpu.SemaphoreType.DMA,
        pltpu.SemaphoreType.DMA,
        pltpu.SemaphoreType.DMA,
        pltpu.SemaphoreType.DMA,
        pltpu.SemaphoreType.DMA,
    ],
)()


def _ffn_body(te_ref, tv_ref, x_ref, w13_ref, w2_ref, ws_ref, y_ref):
    i = pl.program_id(0)

    @pl.when(tv_ref[i] == 1)
    def _():
        xb = x_ref[...]
        g = lax.dot_general(xb, w13_ref[0, :F, :], (((1,), (1,)), ((), ())),
                            preferred_element_type=jnp.float32)
        u = lax.dot_general(xb, w13_ref[0, F:, :], (((1,), (1,)), ((), ())),
                            preferred_element_type=jnp.float32)
        act = jnp.maximum(g, 0.0) * u
        y = lax.dot_general(act, w2_ref[0], (((1,), (1,)), ((), ())),
                            preferred_element_type=jnp.float32)
        y_ref[...] = y * ws_ref[0, 0, :][:, None]


def _ffn(tile_e, tile_v, x_sorted, w13, w2, w_sorted):
    return pl.pallas_call(
        _ffn_body,
        grid_spec=pltpu.PrefetchScalarGridSpec(
            num_scalar_prefetch=2,
            grid=(NT,),
            in_specs=[
                pl.BlockSpec((TM, D), lambda i, te, tv: (i, 0)),
                pl.BlockSpec((1, 2 * F, D), lambda i, te, tv: (te[i], 0, 0)),
                pl.BlockSpec((1, D, F), lambda i, te, tv: (te[i], 0, 0)),
                pl.BlockSpec((1, 1, TM), lambda i, te, tv: (i, 0, 0)),
            ],
            out_specs=pl.BlockSpec((TM, D), lambda i, te, tv: (i, 0)),
        ),
        out_shape=jax.ShapeDtypeStruct((R_CAP, D), jnp.float32),
        compiler_params=pltpu.CompilerParams(
            dimension_semantics=("arbitrary",)),
    )(tile_e, tile_v, x_sorted, w13, w2, w_sorted)


_DCHUNK = 16


_NCH = TPW // _DCHUNK


def _combine_body(y_hbm, pos_hbm, out_hbm, idx0, idx1,
                  r0a, r1a, r0b, r1b, sem_i, g00, g01, g10, g11, sem_o):
    wid = lax.axis_index("s") * NC + lax.axis_index("c")
    base = wid * TPW
    bufs = [(r0a, r1a, g00, g01), (r0b, r1b, g10, g11)]

    cpi0 = pltpu.async_copy(pos_hbm.at[pl.ds(base, TPW)], idx0, sem_i)
    cpi1 = pltpu.async_copy(pos_hbm.at[pl.ds(T + base, TPW)], idx1, sem_i)
    cpi0.wait()
    cpi1.wait()

    def fire(c):
        b0, b1, s0, s1 = bufs[c % 2]
        iv0 = idx0[pl.ds(c * _DCHUNK, 16)]
        iv1 = idx1[pl.ds(c * _DCHUNK, 16)]
        return (pltpu.async_copy(y_hbm.at[iv0], b0, s0),
                pltpu.async_copy(y_hbm.at[iv1], b1, s1))

    desc = {0: fire(0)}
    wbs = {}
    for c in range(_NCH):
        if c + 1 < _NCH:
            if c - 1 >= 0:
                wbs[c - 1].wait()
            desc[c + 1] = fire(c + 1)
        d0, d1 = desc[c]
        d0.wait()
        d1.wait()
        b0, b1 = bufs[c % 2][:2]

        def add_t(t, carry):
            for s in range(D // 16):
                sl = pl.ds(s * 16, 16)
                b0[t, sl] = b0[t, sl] + b1[t, sl]
            return carry

        lax.fori_loop(0, _DCHUNK, add_t, 0)
        wbs[c] = pltpu.async_copy(
            b0, out_hbm.at[pl.ds(base + c * _DCHUNK, _DCHUNK)], sem_o)
    wbs[_NCH - 2].wait()
    wbs[_NCH - 1].wait()


_combine = functools.partial(
    pl.kernel, _combine_body, mesh=_MESH, compiler_params=_SC_PARAMS,
    out_type=jax.ShapeDtypeStruct((T, D), jnp.float32),
    scratch_types=[
        pltpu.VMEM((TPW,), jnp.int32),
        pltpu.VMEM((TPW,), jnp.int32),
        pltpu.VMEM((_DCHUNK, D), jnp.float32),
        pltpu.VMEM((_DCHUNK, D), jnp.float32),
        pltpu.VMEM((_DCHUNK, D), jnp.float32),
        pltpu.VMEM((_DCHUNK, D), jnp.float32),
        pltpu.SemaphoreType.DMA,
        pltpu.SemaphoreType.DMA,
        pltpu.SemaphoreType.DMA,
        pltpu.SemaphoreType.DMA,
        pltpu.SemaphoreType.DMA,
        pltpu.SemaphoreType.DMA,
    ],
)()


def kernel(router_input, hidden_states, Wr, w13, w2):
    logits = _router(Wr, router_input)
    eids, wts, hist = _route(logits)
    x_sorted, w_sorted, pos, tile_e, tile_v = _dispatch(
        eids, wts, hist, hidden_states)
    y = _ffn(tile_e, tile_v, x_sorted, w13, w2,
             w_sorted.reshape(NT, 1, TM))
    return _combine(y, pos)


# speedup vs baseline: 1.6250x; 1.0106x over previous
import functools

import jax
import jax.numpy as jnp
from jax import lax
from jax.experimental import pallas as pl
from jax.experimental.pallas import tpu as pltpu
from jax.experimental.pallas import tpu_sc as plsc

T = 2048
D = 1024
F = 1024
E = 8
K = 2

TM = 256
R_CAP = T * K + E * TM
NT = R_CAP // TM

NC, NS = 2, 16
NW = NC * NS
TPW = T // NW
NG = TPW // 16

_MESH = plsc.VectorSubcoreMesh(core_axis_name="c", subcore_axis_name="s")
_SC_PARAMS = pltpu.CompilerParams(needs_layout_passes=False)


def _lane_iota():
    return lax.iota(jnp.int32, 16)


def _router_body(wr_ref, x_ref, out_ref):
    out_ref[...] = lax.dot_general(
        wr_ref[...], x_ref[...], (((1,), (1,)), ((), ())),
        preferred_element_type=jnp.float32)


def _router(wr, x):
    return pl.pallas_call(
        _router_body,
        out_shape=jax.ShapeDtypeStruct((E, T), jnp.float32),
    )(wr, x)


def _route_body(logits_hbm, eids_hbm, wts_hbm, hist_hbm, lbuf, ebuf0, ebuf1,
                wbuf0, wbuf1, hbuf):
    wid = lax.axis_index("s") * NC + lax.axis_index("c")
    base = wid * TPW
    for e in range(E):
        pltpu.sync_copy(logits_hbm.at[e, pl.ds(base, TPW)], lbuf.at[e])
    lane = _lane_iota()
    hacc = jnp.zeros((16,), jnp.int32)
    for g in range(NG):
        v = [lbuf[e, pl.ds(g * 16, 16)] for e in range(E)]
        m1 = v[0]
        i1 = jnp.zeros((16,), jnp.int32)
        for e in range(1, E):
            upd = v[e] > m1
            m1 = jnp.where(upd, v[e], m1)
            i1 = jnp.where(upd, jnp.int32(e), i1)
        m2 = jnp.full((16,), -jnp.inf, jnp.float32)
        i2 = jnp.zeros((16,), jnp.int32)
        for e in range(E):
            upd = (i1 != e) & (v[e] > m2)
            m2 = jnp.where(upd, v[e], m2)
            i2 = jnp.where(upd, jnp.int32(e), i2)
        dd = jnp.exp(m2 - m1)
        denom = 1.0 + dd
        ebuf0[pl.ds(g * 16, 16)] = i1
        ebuf1[pl.ds(g * 16, 16)] = i2
        wbuf0[pl.ds(g * 16, 16)] = 1.0 / denom
        wbuf1[pl.ds(g * 16, 16)] = dd / denom
        for e in range(E):
            c1 = plsc.all_reduce_population_count(i1 == e)
            c2 = plsc.all_reduce_population_count(i2 == e)
            hacc = hacc + jnp.where(lane == e, c1 + c2, 0)
    hbuf[...] = hacc
    pltpu.sync_copy(ebuf0, eids_hbm.at[pl.ds(base, TPW)])
    pltpu.sync_copy(ebuf1, eids_hbm.at[pl.ds(T + base, TPW)])
    pltpu.sync_copy(wbuf0, wts_hbm.at[pl.ds(base, TPW)])
    pltpu.sync_copy(wbuf1, wts_hbm.at[pl.ds(T + base, TPW)])
    pltpu.sync_copy(hbuf, hist_hbm.at[wid])


_route = functools.partial(
    pl.kernel, _route_body, mesh=_MESH, compiler_params=_SC_PARAMS,
    out_type=[
        jax.ShapeDtypeStruct((K * T,), jnp.int32),
        jax.ShapeDtypeStruct((K * T,), jnp.float32),
        jax.ShapeDtypeStruct((NW, 16), jnp.int32),
    ],
    scratch_types=[
        pltpu.VMEM((E, TPW), jnp.float32),
        pltpu.VMEM((TPW,), jnp.int32),
        pltpu.VMEM((TPW,), jnp.int32),
        pltpu.VMEM((TPW,), jnp.float32),
        pltpu.VMEM((TPW,), jnp.float32),
        pltpu.VMEM((16,), jnp.int32),
    ],
)()


def _dispatch_body(eids_hbm, wts_hbm, hist_hbm, hidden_hbm,
                   xs_hbm, ws_hbm, pos_hbm, te_hbm, tv_hbm,
                   histv, ebuf0, ebuf1, wbuf0, wbuf1, pbuf0, pbuf1,
                   rows_v, tebuf, tvbuf, sem_h, sem_e, sem_r, sem_s, sem_p):
    wid = lax.axis_index("s") * NC + lax.axis_index("c")
    base = wid * TPW
    lane = _lane_iota()

    cp_h = pltpu.async_copy(hist_hbm, histv, sem_h)
    cp_e0 = pltpu.async_copy(eids_hbm.at[pl.ds(base, TPW)], ebuf0, sem_e)
    cp_e1 = pltpu.async_copy(eids_hbm.at[pl.ds(T + base, TPW)], ebuf1, sem_e)
    cp_w0 = pltpu.async_copy(wts_hbm.at[pl.ds(base, TPW)], wbuf0, sem_e)
    cp_w1 = pltpu.async_copy(wts_hbm.at[pl.ds(T + base, TPW)], wbuf1, sem_e)
    cp_r = pltpu.async_copy(hidden_hbm.at[pl.ds(base, TPW)], rows_v, sem_r)

    cp_h.wait()
    tot = jnp.zeros((16,), jnp.int32)
    woff = jnp.zeros((16,), jnp.int32)
    for w in range(NW):
        row = histv[w, :]
        woff = jnp.where(w < wid, woff + row, woff)
        tot = tot + row
    pad_tot = ((tot + (TM - 1)) >> 8) << 8
    gbase = plsc.cumsum(pad_tot) - pad_tot
    bw = gbase + woff

    bw_s = [jnp.sum(jnp.where(lane == e, bw, 0)) for e in range(E)]
    base_s = [jnp.sum(jnp.where(lane == e, gbase, 0)) for e in range(E)]
    tot_s = [jnp.sum(jnp.where(lane == e, tot, 0)) for e in range(E)]

    cp_e0.wait()
    cp_e1.wait()
    carry = [jnp.zeros((16,), jnp.int32) for _ in range(E)]
    for ebuf, pbuf in ((ebuf0, pbuf0), (ebuf1, pbuf1)):
        for g in range(NG):
            evec = ebuf[pl.ds(g * 16, 16)]
            dst = jnp.zeros((16,), jnp.int32)
            for e in range(E):
                mask = evec == e
                rank = plsc.cumsum(mask.astype(jnp.int32)) - 1 + carry[e]
                dst = jnp.where(mask, bw_s[e] + rank, dst)
                carry[e] = carry[e] + plsc.all_reduce_population_count(mask)
            pbuf[pl.ds(g * 16, 16)] = dst

    cp_p0 = pltpu.async_copy(pbuf0, pos_hbm.at[pl.ds(base, TPW)], sem_p)
    cp_p1 = pltpu.async_copy(pbuf1, pos_hbm.at[pl.ds(T + base, TPW)], sem_p)

    cp_r.wait()
    cp_w0.wait()
    cp_w1.wait()
    cp_s0 = pltpu.async_copy(rows_v, xs_hbm.at[pbuf0], sem_s)
    cp_s1 = pltpu.async_copy(rows_v, xs_hbm.at[pbuf1], sem_s)
    cp_s2 = pltpu.async_copy(wbuf0, ws_hbm.at[pbuf0], sem_s)
    cp_s3 = pltpu.async_copy(wbuf1, ws_hbm.at[pbuf1], sem_s)

    @pl.when(wid == 0)
    def _():
        total_rows = jnp.sum(jnp.where(lane < E, pad_tot, 0))
        row_last = total_rows - 1
        c_last = jnp.zeros((), jnp.int32)
        for e in range(E):
            c_last = c_last + jnp.where(base_s[e] <= row_last, 1, 0)
        e_last = c_last - 1
        for grp in range(NT // 16 + (1 if NT % 16 else 0)):
            jv = lane + grp * 16
            rowv = jv * TM
            cj = jnp.zeros((16,), jnp.int32)
            for e in range(E):
                cj = cj + jnp.where(base_s[e] <= rowv, 1, 0)
            ej = cj - 1
            endv = jnp.zeros((16,), jnp.int32)
            for e in range(E):
                endv = endv + jnp.where(ej == e, base_s[e] + tot_s[e], 0)
            validv = rowv < endv
            tev = jnp.where(validv, ej, e_last)
            tebuf[pl.ds(grp * 16, 16)] = tev
            tvbuf[pl.ds(grp * 16, 16)] = validv.astype(jnp.int32)
        pltpu.sync_copy(tebuf, te_hbm)
        pltpu.sync_copy(tvbuf, tv_hbm)

    cp_p0.wait()
    cp_p1.wait()
    cp_s0.wait()
    cp_s1.wait()
    cp_s2.wait()
    cp_s3.wait()


_dispatch = functools.partial(
    pl.kernel, _dispatch_body, mesh=_MESH, compiler_params=_SC_PARAMS,
    out_type=[
        jax.ShapeDtypeStruct((R_CAP, D), jnp.float32),
        jax.ShapeDtypeStruct((R_CAP,), jnp.float32),
        jax.ShapeDtypeStruct((K * T,), jnp.int32),
        jax.ShapeDtypeStruct((32,), jnp.int32),
        jax.ShapeDtypeStruct((32,), jnp.int32),
    ],
    scratch_types=[
        pltpu.VMEM((NW, 16), jnp.int32),
        pltpu.VMEM((TPW,), jnp.int32),
        pltpu.VMEM((TPW,), jnp.int32),
        pltpu.VMEM((TPW,), jnp.float32),
        pltpu.VMEM((TPW,), jnp.float32),
        pltpu.VMEM((TPW,), jnp.int32),
        pltpu.VMEM((TPW,), jnp.int32),
        pltpu.VMEM((TPW, D), jnp.float32),
        pltpu.VMEM((32,), jnp.int32),
        pltpu.VMEM((32,), jnp.int32),
        pltpu.SemaphoreType.DMA,
        pltpu.SemaphoreType.DMA,
        pltpu.SemaphoreType.DMA,
        pltpu.SemaphoreType.DMA,
        pltpu.SemaphoreType.DMA,
    ],
)()


def _ffn_body(te_ref, tv_ref, x_ref, w13_ref, w2_ref, ws_ref, y_ref):
    i = pl.program_id(0)

    @pl.when(tv_ref[i] == 1)
    def _():
        xb = x_ref[...].astype(jnp.bfloat16)
        wg = w13_ref[0, :F, :].astype(jnp.bfloat16)
        wu = w13_ref[0, F:, :].astype(jnp.bfloat16)
        g = lax.dot_general(xb, wg, (((1,), (1,)), ((), ())),
                            preferred_element_type=jnp.float32)
        u = lax.dot_general(xb, wu, (((1,), (1,)), ((), ())),
                            preferred_element_type=jnp.float32)
        act = (jnp.maximum(g, 0.0) * u).astype(jnp.bfloat16)
        w2b = w2_ref[0].astype(jnp.bfloat16)
        y = lax.dot_general(act, w2b, (((1,), (1,)), ((), ())),
                            preferred_element_type=jnp.float32)
        y_ref[...] = y * ws_ref[0, 0, :][:, None]


def _ffn(tile_e, tile_v, x_sorted, w13, w2, w_sorted):
    return pl.pallas_call(
        _ffn_body,
        grid_spec=pltpu.PrefetchScalarGridSpec(
            num_scalar_prefetch=2,
            grid=(NT,),
            in_specs=[
                pl.BlockSpec((TM, D), lambda i, te, tv: (i, 0)),
                pl.BlockSpec((1, 2 * F, D), lambda i, te, tv: (te[i], 0, 0)),
                pl.BlockSpec((1, D, F), lambda i, te, tv: (te[i], 0, 0)),
                pl.BlockSpec((1, 1, TM), lambda i, te, tv: (i, 0, 0)),
            ],
            out_specs=pl.BlockSpec((TM, D), lambda i, te, tv: (i, 0)),
        ),
        out_shape=jax.ShapeDtypeStruct((R_CAP, D), jnp.float32),
        compiler_params=pltpu.CompilerParams(
            dimension_semantics=("arbitrary",)),
    )(tile_e, tile_v, x_sorted, w13, w2, w_sorted)


_DCHUNK = 16


_NCH = TPW // _DCHUNK


def _combine_body(y_hbm, pos_hbm, out_hbm, idx0, idx1,
                  r0a, r1a, r0b, r1b, sem_i, g00, g01, g10, g11, sem_o):
    wid = lax.axis_index("s") * NC + lax.axis_index("c")
    base = wid * TPW
    bufs = [(r0a, r1a, g00, g01), (r0b, r1b, g10, g11)]

    cpi0 = pltpu.async_copy(pos_hbm.at[pl.ds(base, TPW)], idx0, sem_i)
    cpi1 = pltpu.async_copy(pos_hbm.at[pl.ds(T + base, TPW)], idx1, sem_i)
    cpi0.wait()
    cpi1.wait()

    def fire(c):
        b0, b1, s0, s1 = bufs[c % 2]
        iv0 = idx0[pl.ds(c * _DCHUNK, 16)]
        iv1 = idx1[pl.ds(c * _DCHUNK, 16)]
        return (pltpu.async_copy(y_hbm.at[iv0], b0, s0),
                pltpu.async_copy(y_hbm.at[iv1], b1, s1))

    desc = {0: fire(0)}
    wbs = {}
    for c in range(_NCH):
        if c + 1 < _NCH:
            if c - 1 >= 0:
                wbs[c - 1].wait()
            desc[c + 1] = fire(c + 1)
        d0, d1 = desc[c]
        d0.wait()
        d1.wait()
        b0, b1 = bufs[c % 2][:2]

        def add_t(t, carry):
            for s in range(D // 16):
                sl = pl.ds(s * 16, 16)
                b0[t, sl] = b0[t, sl] + b1[t, sl]
            return carry

        lax.fori_loop(0, _DCHUNK, add_t, 0)
        wbs[c] = pltpu.async_copy(
            b0, out_hbm.at[pl.ds(base + c * _DCHUNK, _DCHUNK)], sem_o)
    wbs[_NCH - 2].wait()
    wbs[_NCH - 1].wait()


_combine = functools.partial(
    pl.kernel, _combine_body, mesh=_MESH, compiler_params=_SC_PARAMS,
    out_type=jax.ShapeDtypeStruct((T, D), jnp.float32),
    scratch_types=[
        pltpu.VMEM((TPW,), jnp.int32),
        pltpu.VMEM((TPW,), jnp.int32),
        pltpu.VMEM((_DCHUNK, D), jnp.float32),
        pltpu.VMEM((_DCHUNK, D), jnp.float32),
        pltpu.VMEM((_DCHUNK, D), jnp.float32),
        pltpu.VMEM((_DCHUNK, D), jnp.float32),
        pltpu.SemaphoreType.DMA,
        pltpu.SemaphoreType.DMA,
        pltpu.SemaphoreType.DMA,
        pltpu.SemaphoreType.DMA,
        pltpu.SemaphoreType.DMA,
        pltpu.SemaphoreType.DMA,
    ],
)()


def kernel(router_input, hidden_states, Wr, w13, w2):
    logits = _router(Wr, router_input)
    eids, wts, hist = _route(logits)
    x_sorted, w_sorted, pos, tile_e, tile_v = _dispatch(
        eids, wts, hist, hidden_states)
    y = _ffn(tile_e, tile_v, x_sorted, w13, w2,
             w_sorted.reshape(NT, 1, TM))
    return _combine(y, pos)
